# baseline (speedup 1.0000x reference)
import jax
import jax.numpy as jnp
from jax import lax
from jax.experimental import pallas as pl
from jax.experimental.pallas import tpu as pltpu
from jax.experimental.pallas import tpu_sc as plsc

B, S, D = 4, 8192, 1024
NC, NS = 2, 16
NW = NC * NS
S_PER_W = S // NW
C_ROWS = 16
C = C_ROWS * D
N_CHUNKS = S_PER_W // C_ROWS
EPW = S_PER_W * D


def _sc_body(x_hbm, pos_hbm, out_hbm, posbuf, xbuf):
    wid = lax.axis_index("s") * NC + lax.axis_index("c")
    base = wid * EPW

    def chunk(ci, carry):
        off = base + ci * C
        pltpu.sync_copy(pos_hbm.at[pl.ds(off, C)], posbuf)
        for b in range(B):
            pltpu.sync_copy(x_hbm.at[b, pl.ds(off, C)], xbuf)

            @plsc.parallel_loop(0, C, 16, unroll=8)
            def add_body(i):
                xbuf[pl.ds(i, 16)] = xbuf[pl.ds(i, 16)] + posbuf[pl.ds(i, 16)]

            pltpu.sync_copy(xbuf, out_hbm.at[b, pl.ds(off, C)])
        return carry

    lax.fori_loop(0, N_CHUNKS, chunk, 0)


def kernel(x, pos_table):
    xf = x.reshape(B, S * D)
    posf = pos_table.reshape(-1)
    mesh = plsc.VectorSubcoreMesh(
        core_axis_name="c", subcore_axis_name="s", num_cores=NC, num_subcores=NS
    )
    out = pl.kernel(
        _sc_body,
        out_type=jax.ShapeDtypeStruct((B, S * D), jnp.float32),
        mesh=mesh,
        scratch_types=[
            pltpu.VMEM((C,), jnp.float32),
            pltpu.VMEM((C,), jnp.float32),
        ],
    )(xf, posf)
    return out.reshape(B, S, D)


# speedup vs baseline: 1.2185x; 1.2185x over previous
import jax
import jax.numpy as jnp
from jax import lax
from jax.experimental import pallas as pl
from jax.experimental.pallas import tpu as pltpu
from jax.experimental.pallas import tpu_sc as plsc

B, S, D = 4, 8192, 1024
NC, NS = 2, 16
NW = NC * NS
S_PER_W = S // NW
C_ROWS = 16
C = C_ROWS * D
N_CHUNKS = S_PER_W // C_ROWS
EPW = S_PER_W * D


def _sc_body(x_hbm, pos_hbm, out_hbm,
             posbuf, xb0, xb1, xb2, xb3,
             psem, ls0, ls1, ls2, ls3, ssem):
    wid = lax.axis_index("s") * NC + lax.axis_index("c")
    base = wid * EPW
    xbufs = (xb0, xb1, xb2, xb3)
    lsems = (ls0, ls1, ls2, ls3)

    def chunk(ci, carry):
        off = base + ci * C
        pld = pltpu.async_copy(pos_hbm.at[pl.ds(off, C)], posbuf, psem)
        lds = [
            pltpu.async_copy(x_hbm.at[b, pl.ds(off, C)], xbufs[b], lsems[b])
            for b in range(B)
        ]
        pld.wait()
        sts = []
        for b in range(B):
            lds[b].wait()

            @plsc.parallel_loop(0, C, 16, unroll=8)
            def add_body(i):
                xbufs[b][pl.ds(i, 16)] = (
                    xbufs[b][pl.ds(i, 16)] + posbuf[pl.ds(i, 16)]
                )

            sts.append(
                pltpu.async_copy(xbufs[b], out_hbm.at[b, pl.ds(off, C)], ssem)
            )
        for st in sts:
            st.wait()
        return carry

    lax.fori_loop(0, N_CHUNKS, chunk, 0)


def kernel(x, pos_table):
    xf = x.reshape(B, S * D)
    posf = pos_table.reshape(-1)
    mesh = plsc.VectorSubcoreMesh(
        core_axis_name="c", subcore_axis_name="s", num_cores=NC, num_subcores=NS
    )
    out = pl.kernel(
        _sc_body,
        out_type=jax.ShapeDtypeStruct((B, S * D), jnp.float32),
        mesh=mesh,
        scratch_types=[
            pltpu.VMEM((C,), jnp.float32),
            pltpu.VMEM((C,), jnp.float32),
            pltpu.VMEM((C,), jnp.float32),
            pltpu.VMEM((C,), jnp.float32),
            pltpu.VMEM((C,), jnp.float32),
            pltpu.SemaphoreType.DMA,
            pltpu.SemaphoreType.DMA,
            pltpu.SemaphoreType.DMA,
            pltpu.SemaphoreType.DMA,
            pltpu.SemaphoreType.DMA,
            pltpu.SemaphoreType.DMA,
        ],
    )(xf, posf)
    return out.reshape(B, S, D)


# speedup vs baseline: 5.0650x; 4.1567x over previous
import jax
import jax.numpy as jnp
from jax.experimental import pallas as pl

B, S, D = 4, 8192, 1024
BLK_S = 1024
BLK_B = 2


def _add_kernel(x_ref, pos_ref, out_ref):
    out_ref[...] = x_ref[...] + pos_ref[...][None, :, :]


def kernel(x, pos_table):
    grid = (S // BLK_S, B // BLK_B)
    return pl.pallas_call(
        _add_kernel,
        grid=grid,
        in_specs=[
            pl.BlockSpec((BLK_B, BLK_S, D), lambda i, b: (b, i, 0)),
            pl.BlockSpec((BLK_S, D), lambda i, b: (i, 0)),
        ],
        out_specs=pl.BlockSpec((BLK_B, BLK_S, D), lambda i, b: (b, i, 0)),
        out_shape=jax.ShapeDtypeStruct((B, S, D), x.dtype),
    )(x, pos_table)


# speedup vs baseline: 5.0661x; 1.0002x over previous
import jax
import jax.numpy as jnp
from jax.experimental import pallas as pl

B, S, D = 4, 8192, 1024
BLK_S = 2048


def _add_kernel(x_ref, pos_ref, out_ref):
    out_ref[...] = x_ref[...] + pos_ref[...][None, :, :]


def kernel(x, pos_table):
    grid = (S // BLK_S, B)
    return pl.pallas_call(
        _add_kernel,
        grid=grid,
        in_specs=[
            pl.BlockSpec((1, BLK_S, D), lambda i, b: (b, i, 0)),
            pl.BlockSpec((BLK_S, D), lambda i, b: (i, 0)),
        ],
        out_specs=pl.BlockSpec((1, BLK_S, D), lambda i, b: (b, i, 0)),
        out_shape=jax.ShapeDtypeStruct((B, S, D), x.dtype),
    )(x, pos_table)
